# baseline (speedup 1.0000x reference)
import functools

import jax
import jax.numpy as jnp
from jax import lax
from jax.experimental import pallas as pl
from jax.experimental.pallas import tpu as pltpu
from jax.experimental.pallas import tpu_sc as plsc

_M = 131072
_D = 256
_NW = 32
_PER_W = _M // _NW
_CH = 128
_NCHUNK = _PER_W // _CH


def _sc_body(x_hbm, o_hbm, idx_v, rows_v, sem):
    wid = lax.axis_index("s") * 2 + lax.axis_index("c")
    base = wid * _PER_W
    iota = lax.iota(jnp.int32, 16)
    ixor = jnp.bitwise_xor(iota, 1)

    def chunk(ci, carry):
        start = base + ci * _CH
        for k in range(_CH // 16):
            idx_v[pl.ds(k * 16, 16)] = ixor + (start + k * 16)
        pltpu.async_copy(x_hbm.at[idx_v], rows_v, sem).wait()
        pltpu.sync_copy(rows_v, o_hbm.at[pl.ds(start, _CH)])
        return carry

    lax.fori_loop(0, _NCHUNK, chunk, 0)


_sc_call = functools.partial(
    pl.kernel,
    mesh=plsc.VectorSubcoreMesh(core_axis_name="c", subcore_axis_name="s"),
    out_type=jax.ShapeDtypeStruct((_M, _D), jnp.float32),
    scratch_types=[
        pltpu.VMEM((_CH,), jnp.int32),
        pltpu.VMEM((_CH, _D), jnp.float32),
        pltpu.SemaphoreType.DMA,
    ],
)(_sc_body)


def kernel(x):
    a, b, c, s, t = x.shape
    g, l = c // 128, 128
    xr = (
        x.transpose(0, 1, 3, 4, 2)
        .reshape(a, b, s, t, g, l)
        .transpose(0, 1, 2, 4, 3, 5)
        .reshape(_M, _D)
    )
    out = _sc_call(xr)
    return (
        out.reshape(a, b, s, g, t, l)
        .transpose(0, 1, 2, 4, 3, 5)
        .reshape(a, b, s, t, c)
        .transpose(0, 1, 4, 2, 3)
    )


# speedup vs baseline: 1.0305x; 1.0305x over previous
import functools

import jax
import jax.numpy as jnp
from jax import lax
from jax.experimental import pallas as pl
from jax.experimental.pallas import tpu as pltpu
from jax.experimental.pallas import tpu_sc as plsc

_M = 131072
_D = 256
_NW = 32
_PER_W = _M // _NW
_CH = 128
_NCHUNK = _PER_W // _CH


def _sc_body(x_hbm, o_hbm, idx_v, buf0, buf1, sg0, sg1, ss0, ss1):
    wid = lax.axis_index("s") * 2 + lax.axis_index("c")
    base = wid * _PER_W
    iota = lax.iota(jnp.int32, 16)
    ixor = jnp.bitwise_xor(iota, 1)

    def build(k, carry):
        idx_v[pl.ds(k * 16, 16)] = ixor + (base + k * 16)
        return carry

    lax.fori_loop(0, _PER_W // 16, build, 0)

    bufs = (buf0, buf1)
    sgs = (sg0, sg1)
    sss = (ss0, ss1)

    def gather(c, b):
        pltpu.async_copy(x_hbm.at[idx_v.at[pl.ds(c * _CH, _CH)]], bufs[b], sgs[b])

    def scatter(c, b):
        pltpu.async_copy(bufs[b], o_hbm.at[pl.ds(base + c * _CH, _CH)], sss[b])

    gather(0, 0)
    gather(1, 1)

    def pair(i, carry):
        c0 = 2 * i
        for b in range(2):
            c = c0 + b
            pltpu.make_async_copy(
                x_hbm.at[idx_v.at[pl.ds(0, _CH)]], bufs[b], sgs[b]
            ).wait()
            scatter(c, b)
        for b in range(2):
            c2 = c0 + 2 + b
            pltpu.make_async_copy(
                bufs[b], o_hbm.at[pl.ds(base, _CH)], sss[b]
            ).wait()

            @pl.when(c2 < _NCHUNK)
            def _():
                gather(c2, b)

        return carry

    lax.fori_loop(0, _NCHUNK // 2, pair, 0)


_sc_call = functools.partial(
    pl.kernel,
    mesh=plsc.VectorSubcoreMesh(core_axis_name="c", subcore_axis_name="s"),
    out_type=jax.ShapeDtypeStruct((_M, _D), jnp.float32),
    scratch_types=[
        pltpu.VMEM((_PER_W,), jnp.int32),
        pltpu.VMEM((_CH, _D), jnp.float32),
        pltpu.VMEM((_CH, _D), jnp.float32),
        pltpu.SemaphoreType.DMA,
        pltpu.SemaphoreType.DMA,
        pltpu.SemaphoreType.DMA,
        pltpu.SemaphoreType.DMA,
    ],
)(_sc_body)


def kernel(x):
    a, b, c, s, t = x.shape
    g, l = c // 128, 128
    xr = (
        x.transpose(0, 1, 3, 4, 2)
        .reshape(a, b, s, t, g, l)
        .transpose(0, 1, 2, 4, 3, 5)
        .reshape(_M, _D)
    )
    out = _sc_call(xr)
    return (
        out.reshape(a, b, s, g, t, l)
        .transpose(0, 1, 2, 4, 3, 5)
        .reshape(a, b, s, t, c)
        .transpose(0, 1, 4, 2, 3)
    )


# speedup vs baseline: 1.1932x; 1.1579x over previous
import functools

import jax
import jax.numpy as jnp
from jax import lax
from jax.experimental import pallas as pl
from jax.experimental.pallas import tpu as pltpu
from jax.experimental.pallas import tpu_sc as plsc

_A = 64
_A0 = 12
_ROWS_PER_A = 4096

_D = 256
_NW = 32
_M = 131072
_M_SC = _A0 * 2048
_PER_W = _M_SC // _NW
_CH = 128
_NCHUNK = _PER_W // _CH

_LANES = 128
_ROWS = 262144
_ROWS_TC = _ROWS - _A0 * _ROWS_PER_A
_BLOCK_ROWS = 16384


def _sc_body(x_hbm, o_hbm, idx_v, buf0, buf1, sg0, sg1, ss0, ss1):
    wid = lax.axis_index("s") * 2 + lax.axis_index("c")
    base = wid * _PER_W
    iota = lax.iota(jnp.int32, 16)
    ixor = jnp.bitwise_xor(iota, 1)

    def build(k, carry):
        idx_v[pl.ds(k * 16, 16)] = ixor + (base + k * 16)
        return carry

    lax.fori_loop(0, _PER_W // 16, build, 0)

    bufs = (buf0, buf1)
    sgs = (sg0, sg1)
    sss = (ss0, ss1)

    def gather(c, b):
        pltpu.async_copy(x_hbm.at[idx_v.at[pl.ds(c * _CH, _CH)]], bufs[b], sgs[b])

    def scatter(c, b):
        pltpu.async_copy(bufs[b], o_hbm.at[pl.ds(base + c * _CH, _CH)], sss[b])

    gather(0, 0)
    gather(1, 1)

    def pair(i, carry):
        c0 = 2 * i
        for b in range(2):
            c = c0 + b
            pltpu.make_async_copy(
                x_hbm.at[idx_v.at[pl.ds(0, _CH)]], bufs[b], sgs[b]
            ).wait()
            scatter(c, b)
        for b in range(2):
            c2 = c0 + 2 + b
            pltpu.make_async_copy(
                bufs[b], o_hbm.at[pl.ds(base, _CH)], sss[b]
            ).wait()

            @pl.when(c2 < _NCHUNK)
            def _():
                gather(c2, b)

        return carry

    lax.fori_loop(0, _NCHUNK // 2, pair, 0)


_sc_call = functools.partial(
    pl.kernel,
    mesh=plsc.VectorSubcoreMesh(core_axis_name="c", subcore_axis_name="s"),
    out_type=jax.ShapeDtypeStruct((_M_SC, _D), jnp.float32),
    scratch_types=[
        pltpu.VMEM((_PER_W,), jnp.int32),
        pltpu.VMEM((_CH, _D), jnp.float32),
        pltpu.VMEM((_CH, _D), jnp.float32),
        pltpu.SemaphoreType.DMA,
        pltpu.SemaphoreType.DMA,
        pltpu.SemaphoreType.DMA,
        pltpu.SemaphoreType.DMA,
    ],
)(_sc_body)


def _tc_body(x_ref, o_ref):
    v = x_ref[...]
    sub = jax.lax.broadcasted_iota(jnp.int32, v.shape, 0)
    fwd = pltpu.roll(v, v.shape[0] - 2, 0)
    bwd = pltpu.roll(v, 2, 0)
    o_ref[...] = jnp.where((sub & 2) == 0, fwd, bwd)


def _tc_call(xr):
    off = _A0 * _ROWS_PER_A // _BLOCK_ROWS
    return pl.pallas_call(
        _tc_body,
        grid=(_ROWS_TC // _BLOCK_ROWS,),
        in_specs=[pl.BlockSpec((_BLOCK_ROWS, _LANES), lambda i: (i + off, 0))],
        out_specs=pl.BlockSpec((_BLOCK_ROWS, _LANES), lambda i: (i, 0)),
        out_shape=jax.ShapeDtypeStruct((_ROWS_TC, _LANES), jnp.float32),
    )(xr)


def kernel(x):
    a, b, c, s, t = x.shape
    g, l = c // 128, 128
    xr = (
        x.transpose(0, 1, 3, 4, 2)
        .reshape(a, b, s, t, g, l)
        .transpose(0, 1, 2, 4, 3, 5)
        .reshape(_ROWS, _LANES)
    )
    sc_out = _sc_call(xr.reshape(_M, _D))
    tc_out = _tc_call(xr)
    out = jnp.concatenate([sc_out.reshape(_M_SC * 2, _LANES), tc_out], axis=0)
    return (
        out.reshape(a, b, s, g, t, l)
        .transpose(0, 1, 2, 4, 3, 5)
        .reshape(a, b, s, t, c)
        .transpose(0, 1, 4, 2, 3)
    )
